# baseline (speedup 1.0000x reference)
import functools

import numpy as np

import jax
import jax.numpy as jnp
from jax import lax
from jax.experimental import pallas as pl
from jax.experimental.pallas import tpu as pltpu
from jax.experimental.pallas import tpu_sc as plsc

N = 10000
E = 320000
FEAT = 128
HID = 128
HEADS = 8
G = 16
LANES = 16
NC = 2
NS = 16
NW = NC * NS
EPW = E // NW
C2 = 16
G2 = EPW // C2
C1 = 40
G1 = EPW // C1
RPT = 624
RPT_LAST = N - (NS - 1) * RPT


def _split_copy(sid, mk_src, mk_dst):
  off = sid * RPT

  @pl.when(sid < NS - 1)
  def _():
    pltpu.sync_copy(mk_src(off, RPT), mk_dst(off, RPT))

  @pl.when(sid == NS - 1)
  def _():
    pltpu.sync_copy(mk_src(off, RPT_LAST), mk_dst(off, RPT_LAST))

_SC_MESH = plsc.VectorSubcoreMesh(
    core_axis_name="c", subcore_axis_name="s", num_cores=NC, num_subcores=NS)

_BCAST_DNUMS = lax.GatherDimensionNumbers(
    offset_dims=(), collapsed_slice_dims=(0,), start_index_map=(0,))


def _lane_bcast(vec, lane):
  idx = jnp.full((LANES, 1), lane, jnp.int32)
  return lax.gather(vec, idx, _BCAST_DNUMS, (1,),
                    mode=lax.GatherScatterMode.PROMISE_IN_BOUNDS)


def _mm_body(x_ref, w_ref, t_ref, wlo_ref, whi_ref, h_ref, tout_ref,
             hpk_ref):
  x = x_ref[...]
  h = jnp.dot(x, w_ref[...], preferred_element_type=jnp.float32)
  h_ref[...] = h
  tout_ref[...] = jnp.dot(h, t_ref[...], preferred_element_type=jnp.float32)
  ha = jnp.dot(x, wlo_ref[...], preferred_element_type=jnp.float32)
  hb = jnp.dot(x, whi_ref[...], preferred_element_type=jnp.float32)
  au = lax.bitcast_convert_type(
      lax.convert_element_type(ha, jnp.bfloat16), jnp.uint16)
  bu = lax.bitcast_convert_type(
      lax.convert_element_type(hb, jnp.bfloat16), jnp.uint16)
  word = au.astype(jnp.uint32) | (bu.astype(jnp.uint32) << 16)
  blk = word.shape[0]
  hpk_ref[...] = lax.bitcast_convert_type(word, jnp.int32).reshape(
      blk, 4, HID)


def _mm(x, w, tmat, wlo, whi):
  n, k = x.shape
  m = w.shape[1]
  blk = 400
  return pl.pallas_call(
      _mm_body,
      grid=(n // blk,),
      in_specs=[
          pl.BlockSpec((blk, k), lambda i: (i, 0)),
          pl.BlockSpec((k, m), lambda i: (0, 0)),
          pl.BlockSpec((m, HID), lambda i: (0, 0)),
          pl.BlockSpec((k, m // 2), lambda i: (0, 0)),
          pl.BlockSpec((k, m // 2), lambda i: (0, 0)),
      ],
      out_specs=[
          pl.BlockSpec((blk, m), lambda i: (i, 0)),
          pl.BlockSpec((blk, HID), lambda i: (i, 0)),
          pl.BlockSpec((blk, 4, HID), lambda i: (i, 0, 0)),
      ],
      out_shape=[
          jax.ShapeDtypeStruct((n, m), jnp.float32),
          jax.ShapeDtypeStruct((n, HID), jnp.float32),
          jax.ShapeDtypeStruct((n, 4, HID), jnp.int32),
      ],
  )(x, w, tmat, wlo, whi)


def _sc_pass1_body(t_hbm, src_hbm, dst_hbm, z_hbm,
                   ex_hbm, dpart_hbm,
                   sidxb, didxb, dstc, tsb, tdb, exb, exs, acc,
                   isem0, isem1, gsem0, gsem1, wsem0, wsem1):
  cid = lax.axis_index("c")
  sid = lax.axis_index("s")
  wid = sid * NC + cid
  isems = (isem0, isem1)
  gsems = (gsem0, gsem1)
  wsems = (wsem0, wsem1)

  _split_copy(sid,
              lambda o, n: z_hbm.at[pl.ds(o, n)],
              lambda o, n: acc.at[pl.ds(o, n)])
  pltpu.sync_copy(z_hbm.at[pl.ds(0, C1)], exs)
  plsc.subcore_barrier()

  ebase = wid * EPW

  rot_idx = jnp.reshape((lax.iota(jnp.int32, LANES) + 8) & 15, (LANES, 1))

  def issue_idx(g, slot):
    pltpu.async_copy(src_hbm.at[pl.ds(ebase + g * C1, C1)], sidxb.at[slot],
                     isems[slot])
    pltpu.async_copy(dst_hbm.at[pl.ds(ebase + g * C1, C1)], didxb.at[slot],
                     isems[slot])

  def wait_idx(slot):
    pltpu.make_async_copy(
        src_hbm.at[pl.ds(0, C1)], sidxb.at[slot], isems[slot]).wait()
    pltpu.make_async_copy(
        src_hbm.at[pl.ds(0, C1)], didxb.at[slot], isems[slot]).wait()

  def issue_data(slot):
    pltpu.async_copy(t_hbm.at[sidxb.at[slot]], tsb.at[slot], gsems[slot])
    pltpu.async_copy(t_hbm.at[didxb.at[slot]], tdb.at[slot], gsems[slot])

  def wait_data(slot):
    pltpu.make_async_copy(
        t_hbm.at[pl.ds(0, C1)], tsb.at[slot], gsems[slot]).wait()
    pltpu.make_async_copy(
        t_hbm.at[pl.ds(0, C1)], tdb.at[slot], gsems[slot]).wait()

  def drain_write(slot):
    pltpu.make_async_copy(
        ex_hbm.at[pl.ds(0, C1)], exb.at[slot], wsems[slot]).wait()

  def compute(g, slot):
    @pl.when(g >= 2)
    def _():
      drain_write(slot)
    def edge(j, c):
      vs = tsb[slot, j, pl.ds(0, LANES)]
      vd = tdb[slot, j, pl.ds(0, LANES)]
      vdr = lax.gather(vd, rot_idx, _BCAST_DNUMS, (1,),
                       mode=lax.GatherScatterMode.PROMISE_IN_BOUNDS)
      al = vs + vdr
      al = jnp.maximum(al, al * jnp.float32(0.2))
      ex = jnp.exp(al)
      exb[slot, j] = ex
      exs[j, pl.ds(0, LANES)] = ex
      return c

    lax.fori_loop(0, C1, edge, 0)
    for off in (0, 16, 24):
      dstc[pl.ds(off, LANES)] = didxb[slot, pl.ds(off, LANES)]
    pltpu.sync_copy(exs, acc.at[dstc], add=True)
    pltpu.async_copy(exb.at[slot],
                     ex_hbm.at[pl.ds(ebase + g * C1, C1)], wsems[slot])

  issue_idx(0, 0)
  issue_idx(1, 1)
  wait_idx(0)
  issue_data(0)

  def body(i, carry):
    g0 = 2 * i
    wait_idx(1)
    issue_data(1)
    wait_data(0)
    compute(g0, 0)
    issue_idx(jnp.minimum(g0 + 2, G1 - 1), 0)
    wait_idx(0)
    issue_data(0)
    wait_data(1)
    compute(g0 + 1, 1)
    issue_idx(jnp.minimum(g0 + 3, G1 - 1), 1)
    return carry

  lax.fori_loop(0, G1 // 2, body, 0)
  wait_data(0)
  wait_idx(1)
  drain_write(0)
  drain_write(1)

  plsc.subcore_barrier()
  _split_copy(sid,
              lambda o, n: acc.at[pl.ds(o, n)],
              lambda o, n: dpart_hbm.at[cid, pl.ds(o, n)])


_sc_pass1 = pl.kernel(
    _sc_pass1_body,
    out_type=(jax.ShapeDtypeStruct((E, LANES), jnp.float32),
              jax.ShapeDtypeStruct((NC, N, HID), jnp.float32)),
    mesh=_SC_MESH,
    scratch_types=[
        pltpu.VMEM((2, C1), jnp.int32),
        pltpu.VMEM((2, C1), jnp.int32),
        pltpu.VMEM((C1,), jnp.int32),
        pltpu.VMEM((2, C1, HID), jnp.float32),
        pltpu.VMEM((2, C1, HID), jnp.float32),
        pltpu.VMEM((2, C1, LANES), jnp.float32),
        pltpu.VMEM((C1, HID), jnp.float32),
        pltpu.VMEM_SHARED((N, HID), jnp.float32),
        pltpu.SemaphoreType.DMA,
        pltpu.SemaphoreType.DMA,
        pltpu.SemaphoreType.DMA,
        pltpu.SemaphoreType.DMA,
        pltpu.SemaphoreType.DMA,
        pltpu.SemaphoreType.DMA,
    ],
)


def _dinv_body(d0_ref, d1_ref, t_ref, dinv_ref, cs_ref):
  t = t_ref[...]
  al = t[:, 0:HEADS] + t[:, HEADS:2 * HEADS]
  al = jnp.maximum(al, al * jnp.float32(0.2))
  exs = jnp.exp(al)
  den = d0_ref[...][:, 0:HEADS] + d1_ref[...][:, 0:HEADS] + exs
  dv = 1.0 / (den + jnp.float32(1e-16))
  blk = t.shape[0]
  dinv_ref[...] = jnp.concatenate(
      [dv, jnp.zeros((blk, HID - HEADS), jnp.float32)], axis=1)
  cs_ref[...] = exs * dv


def _dinv(d0, d1, t):
  blk = 2000
  spec = pl.BlockSpec((blk, HID), lambda i: (i, 0))
  return pl.pallas_call(
      _dinv_body,
      grid=(N // blk,),
      in_specs=[spec, spec, spec],
      out_specs=[spec, pl.BlockSpec((blk, HEADS), lambda i: (i, 0))],
      out_shape=[jax.ShapeDtypeStruct((N, HID), jnp.float32),
                 jax.ShapeDtypeStruct((N, HEADS), jnp.float32)],
  )(d0, d1, t)


def _sc_pass2_body(h_hbm, dinv_hbm, ex_hbm, src_hbm, dst_hbm, z_hbm,
                   mpart_hbm,
                   sidxb, didxb, dstc, hbuf, exb, dvb, msgb, acc,
                   isem0, isem1, hsem0, hsem1, esem0, esem1):
  cid = lax.axis_index("c")
  sid = lax.axis_index("s")
  wid = sid * NC + cid
  isems = (isem0, isem1)
  hsems = (hsem0, hsem1)
  esems = (esem0, esem1)

  _split_copy(sid,
              lambda o, n: z_hbm.at[pl.ds(o, n)],
              lambda o, n: acc.at[pl.ds(o, n)])
  plsc.subcore_barrier()

  ebase = wid * EPW

  def issue_idx(g, slot):
    pltpu.async_copy(src_hbm.at[pl.ds(ebase + g * C2, C2)], sidxb.at[slot],
                     isems[slot])
    pltpu.async_copy(dst_hbm.at[pl.ds(ebase + g * C2, C2)], didxb.at[slot],
                     isems[slot])

  def wait_idx(slot):
    pltpu.make_async_copy(
        src_hbm.at[pl.ds(0, C2)], sidxb.at[slot], isems[slot]).wait()
    pltpu.make_async_copy(
        src_hbm.at[pl.ds(0, C2)], didxb.at[slot], isems[slot]).wait()

  def issue_data(g, slot):
    pltpu.async_copy(h_hbm.at[sidxb.at[slot]], hbuf.at[slot], hsems[slot])
    pltpu.async_copy(dinv_hbm.at[didxb.at[slot]], dvb.at[slot], esems[slot])
    pltpu.async_copy(ex_hbm.at[pl.ds(ebase + g * C2, C2)], exb.at[slot],
                     esems[slot])

  def wait_data(slot):
    pltpu.make_async_copy(
        h_hbm.at[pl.ds(0, C2)], hbuf.at[slot], hsems[slot]).wait()
    pltpu.make_async_copy(
        dinv_hbm.at[pl.ds(0, C2)], dvb.at[slot], esems[slot]).wait()
    pltpu.make_async_copy(
        ex_hbm.at[pl.ds(0, C2)], exb.at[slot], esems[slot]).wait()

  def compute(g, slot):
    for j in range(C2):
      coef = exb[slot, j] * dvb[slot, j, pl.ds(0, LANES)]
      m = [None] * 8
      for hh in range(HEADS):
        c = _lane_bcast(coef, hh)
        for t2 in range(4):
          wi = (hh * 4 + t2) * LANES
          w = hbuf[slot, j, wi // HID, pl.ds(wi % HID, LANES)]
          va = lax.bitcast_convert_type(lax.shift_left(w, 16), jnp.float32)
          vb = lax.bitcast_convert_type(w & jnp.int32(-65536), jnp.float32)
          k = 2 * t2
          m[k] = c * va if hh == 0 else m[k] + c * va
          m[k + 1] = c * vb if hh == 0 else m[k + 1] + c * vb
      for k in range(8):
        msgb[j, pl.ds(k * LANES, LANES)] = m[k]
    dstc[pl.ds(0, C2)] = didxb[slot, pl.ds(0, C2)]
    pltpu.sync_copy(msgb, acc.at[dstc], add=True)

  issue_idx(0, 0)
  issue_idx(1, 1)
  wait_idx(0)
  issue_data(0, 0)

  def body(i, carry):
    g0 = 2 * i
    wait_idx(1)
    issue_data(g0 + 1, 1)
    wait_data(0)
    compute(g0, 0)
    issue_idx(jnp.minimum(g0 + 2, G2 - 1), 0)
    wait_idx(0)
    issue_data(jnp.minimum(g0 + 2, G2 - 1), 0)
    wait_data(1)
    compute(g0 + 1, 1)
    issue_idx(jnp.minimum(g0 + 3, G2 - 1), 1)
    return carry

  lax.fori_loop(0, G2 // 2, body, 0)
  wait_data(0)
  compute(G2 - 1, 0)
  wait_idx(1)

  plsc.subcore_barrier()
  _split_copy(sid,
              lambda o, n: acc.at[pl.ds(o, n)],
              lambda o, n: mpart_hbm.at[cid, pl.ds(o, n)])


_sc_pass2 = pl.kernel(
    _sc_pass2_body,
    out_type=jax.ShapeDtypeStruct((NC, N, HID), jnp.float32),
    mesh=_SC_MESH,
    scratch_types=[
        pltpu.VMEM((2, C2), jnp.int32),
        pltpu.VMEM((2, C2), jnp.int32),
        pltpu.VMEM((C2,), jnp.int32),
        pltpu.VMEM((2, C2, 4, HID), jnp.int32),
        pltpu.VMEM((2, C2, LANES), jnp.float32),
        pltpu.VMEM((2, C2, HID), jnp.float32),
        pltpu.VMEM((C2, HID), jnp.float32),
        pltpu.VMEM_SHARED((N, HID), jnp.float32),
        pltpu.SemaphoreType.DMA,
        pltpu.SemaphoreType.DMA,
        pltpu.SemaphoreType.DMA,
        pltpu.SemaphoreType.DMA,
        pltpu.SemaphoreType.DMA,
        pltpu.SemaphoreType.DMA,
    ],
)


def _fin_body(m0_ref, m1_ref, h_ref, cs_ref, b_ref, y_ref):
  blk = h_ref.shape[0]
  hb = h_ref[...].reshape(blk, HEADS, HID)
  cs = cs_ref[...]
  selfterm = (hb * cs[:, :, None]).sum(axis=1)
  y = (m0_ref[...] + m1_ref[...] + selfterm) * jnp.float32(1.0 / HEADS)
  y = y + b_ref[...]
  y_ref[...] = jnp.where(y > 0, y, jnp.exp(y) - 1.0)


def _fin(m0, m1, h, cs, b2d):
  blk = 400
  return pl.pallas_call(
      _fin_body,
      grid=(N // blk,),
      in_specs=[
          pl.BlockSpec((blk, HID), lambda i: (i, 0)),
          pl.BlockSpec((blk, HID), lambda i: (i, 0)),
          pl.BlockSpec((blk, HEADS * HID), lambda i: (i, 0)),
          pl.BlockSpec((blk, HEADS), lambda i: (i, 0)),
          pl.BlockSpec((1, HID), lambda i: (0, 0)),
      ],
      out_specs=pl.BlockSpec((blk, HID), lambda i: (i, 0)),
      out_shape=jax.ShapeDtypeStruct((N, HID), jnp.float32),
  )(m0, m1, h, cs, b2d)


def _pool_body(b_ref, y0_ref, y1_ref, y2_ref, o0_ref, o1_ref, o2_ref):
  i = pl.program_id(0)

  @pl.when(i == 0)
  def _():
    o0_ref[...] = jnp.zeros_like(o0_ref)
    o1_ref[...] = jnp.zeros_like(o1_ref)
    o2_ref[...] = jnp.zeros_like(o2_ref)

  b = b_ref[0, 0, :]
  blk = b.shape[0]
  onehot = (lax.broadcasted_iota(jnp.int32, (G, blk), 0)
            == b[None, :]).astype(jnp.float32)
  o0_ref[...] += jnp.dot(onehot, y0_ref[...],
                         preferred_element_type=jnp.float32)
  o1_ref[...] += jnp.dot(onehot, y1_ref[...],
                         preferred_element_type=jnp.float32)
  o2_ref[...] += jnp.dot(onehot, y2_ref[...],
                         preferred_element_type=jnp.float32)


def _pool(batch_r, y0, y1, y2):
  blk = 400
  yspec = pl.BlockSpec((blk, HID), lambda i: (i, 0))
  ospec = pl.BlockSpec((G, HID), lambda i: (0, 0))
  oshape = jax.ShapeDtypeStruct((G, HID), jnp.float32)
  return pl.pallas_call(
      _pool_body,
      grid=(N // blk,),
      in_specs=[pl.BlockSpec((1, 1, blk), lambda i: (i, 0, 0)),
                yspec, yspec, yspec],
      out_specs=[ospec, ospec, ospec],
      out_shape=[oshape, oshape, oshape],
  )(batch_r, y0, y1, y2)


def _tmat(att_s, att_d):
  eye = jnp.eye(HEADS, dtype=jnp.float32)
  ms = (att_s[0][:, :, None] * eye[:, None, :]).reshape(HEADS * HID, HEADS)
  md = (att_d[0][:, :, None] * eye[:, None, :]).reshape(HEADS * HID, HEADS)
  return jnp.pad(jnp.concatenate([ms, md], axis=1),
                 ((0, 0), (0, HID - 2 * HEADS)))


def kernel(x, edge_index, batch, edge_attr,
           W0, att_src0, att_dst0, bias0,
           W1, att_src1, att_dst1, bias1,
           W2, att_src2, att_dst2, bias2):
  del edge_attr
  src_r = edge_index[0].astype(jnp.int32)
  dst_r = edge_index[1].astype(jnp.int32)
  batch_r = batch.astype(jnp.int32).reshape(N // 400, 1, 400)
  z128 = jnp.zeros((N, HID), jnp.float32)

  params = [(W0, att_src0, att_dst0, bias0),
            (W1, att_src1, att_dst1, bias1),
            (W2, att_src2, att_dst2, bias2)]
  wi = np.arange(HEADS * HID // 2)
  t_blk, i_lane = wi // LANES, wi % LANES
  perm_lo = jnp.asarray(32 * t_blk + i_lane, jnp.int32)
  perm_hi = jnp.asarray(32 * t_blk + LANES + i_lane, jnp.int32)
  h_in = x
  ys = []
  for (W, a_s, a_d, b) in params:
    h, t, hpk = _mm(h_in, W, _tmat(a_s, a_d), W[:, perm_lo], W[:, perm_hi])
    ex, dpart = _sc_pass1(t, src_r, dst_r, z128)
    dinv, cself = _dinv(dpart[0], dpart[1], t)
    mpart = _sc_pass2(hpk, dinv, ex, src_r, dst_r, z128)
    y = _fin(mpart[0], mpart[1], h, cself, b.reshape(1, HID))
    ys.append(y)
    h_in = y

  rep0, rep1, rep2 = _pool(batch_r, ys[0], ys[1], ys[2])
  global_rep = jnp.concatenate([rep0, rep1, rep2], axis=1)
  return (global_rep, h_in)


# speedup vs baseline: 1.4727x; 1.4727x over previous
import functools

import numpy as np

import jax
import jax.numpy as jnp
from jax import lax
from jax.experimental import pallas as pl
from jax.experimental.pallas import tpu as pltpu
from jax.experimental.pallas import tpu_sc as plsc

N = 10000
E = 320000
FEAT = 128
HID = 128
HEADS = 8
G = 16
LANES = 16
NC = 2
NS = 16
NW = NC * NS
EPW = E // NW
C2 = 16
G2 = EPW // C2
C1 = 40
G1 = EPW // C1
RPT = 624
RPT_LAST = N - (NS - 1) * RPT


def _split_copy(sid, mk_src, mk_dst):
  off = sid * RPT

  @pl.when(sid < NS - 1)
  def _():
    pltpu.sync_copy(mk_src(off, RPT), mk_dst(off, RPT))

  @pl.when(sid == NS - 1)
  def _():
    pltpu.sync_copy(mk_src(off, RPT_LAST), mk_dst(off, RPT_LAST))

_SC_MESH = plsc.VectorSubcoreMesh(
    core_axis_name="c", subcore_axis_name="s", num_cores=NC, num_subcores=NS)

_BCAST_DNUMS = lax.GatherDimensionNumbers(
    offset_dims=(), collapsed_slice_dims=(0,), start_index_map=(0,))


def _lane_bcast(vec, lane):
  idx = jnp.full((LANES, 1), lane, jnp.int32)
  return lax.gather(vec, idx, _BCAST_DNUMS, (1,),
                    mode=lax.GatherScatterMode.PROMISE_IN_BOUNDS)


def _mm_body(x_ref, w_ref, t_ref, wlo_ref, whi_ref, h_ref, tout_ref,
             hpk_ref):
  x = x_ref[...]
  h = jnp.dot(x, w_ref[...], preferred_element_type=jnp.float32)
  h_ref[...] = h
  tout_ref[...] = jnp.dot(h, t_ref[...], preferred_element_type=jnp.float32)
  ha = jnp.dot(x, wlo_ref[...], preferred_element_type=jnp.float32)
  hb = jnp.dot(x, whi_ref[...], preferred_element_type=jnp.float32)
  au = lax.bitcast_convert_type(
      lax.convert_element_type(ha, jnp.bfloat16), jnp.uint16)
  bu = lax.bitcast_convert_type(
      lax.convert_element_type(hb, jnp.bfloat16), jnp.uint16)
  word = au.astype(jnp.uint32) | (bu.astype(jnp.uint32) << 16)
  blk = word.shape[0]
  hpk_ref[...] = lax.bitcast_convert_type(word, jnp.int32).reshape(
      blk, 4, HID)


def _mm(x, w, tmat, wlo, whi):
  n, k = x.shape
  m = w.shape[1]
  blk = 400
  return pl.pallas_call(
      _mm_body,
      grid=(n // blk,),
      in_specs=[
          pl.BlockSpec((blk, k), lambda i: (i, 0)),
          pl.BlockSpec((k, m), lambda i: (0, 0)),
          pl.BlockSpec((m, HID), lambda i: (0, 0)),
          pl.BlockSpec((k, m // 2), lambda i: (0, 0)),
          pl.BlockSpec((k, m // 2), lambda i: (0, 0)),
      ],
      out_specs=[
          pl.BlockSpec((blk, m), lambda i: (i, 0)),
          pl.BlockSpec((blk, HID), lambda i: (i, 0)),
          pl.BlockSpec((blk, 4, HID), lambda i: (i, 0, 0)),
      ],
      out_shape=[
          jax.ShapeDtypeStruct((n, m), jnp.float32),
          jax.ShapeDtypeStruct((n, HID), jnp.float32),
          jax.ShapeDtypeStruct((n, 4, HID), jnp.int32),
      ],
  )(x, w, tmat, wlo, whi)


def _sc_pass1_body(t_hbm, src_hbm, dst_hbm, z_hbm,
                   ex_hbm, dpart_hbm,
                   sidxb, didxb, dstc, tsb, tdb, exb, exs, acc,
                   isem0, isem1, gsem0, gsem1, wsem0, wsem1):
  cid = lax.axis_index("c")
  sid = lax.axis_index("s")
  wid = sid * NC + cid
  isems = (isem0, isem1)
  gsems = (gsem0, gsem1)
  wsems = (wsem0, wsem1)

  _split_copy(sid,
              lambda o, n: z_hbm.at[pl.ds(o, n)],
              lambda o, n: acc.at[pl.ds(o, n)])
  pltpu.sync_copy(z_hbm.at[pl.ds(0, C1)], exs)
  plsc.subcore_barrier()

  ebase = wid * EPW

  rot_idx = jnp.reshape((lax.iota(jnp.int32, LANES) + 8) & 15, (LANES, 1))

  def issue_idx(g, slot):
    pltpu.async_copy(src_hbm.at[pl.ds(ebase + g * C1, C1)], sidxb.at[slot],
                     isems[slot])
    pltpu.async_copy(dst_hbm.at[pl.ds(ebase + g * C1, C1)], didxb.at[slot],
                     isems[slot])

  def wait_idx(slot):
    pltpu.make_async_copy(
        src_hbm.at[pl.ds(0, C1)], sidxb.at[slot], isems[slot]).wait()
    pltpu.make_async_copy(
        src_hbm.at[pl.ds(0, C1)], didxb.at[slot], isems[slot]).wait()

  def issue_data(slot):
    pltpu.async_copy(t_hbm.at[sidxb.at[slot]], tsb.at[slot], gsems[slot])
    pltpu.async_copy(t_hbm.at[didxb.at[slot]], tdb.at[slot], gsems[slot])

  def wait_data(slot):
    pltpu.make_async_copy(
        t_hbm.at[pl.ds(0, C1)], tsb.at[slot], gsems[slot]).wait()
    pltpu.make_async_copy(
        t_hbm.at[pl.ds(0, C1)], tdb.at[slot], gsems[slot]).wait()

  def drain_write(slot):
    pltpu.make_async_copy(
        ex_hbm.at[pl.ds(0, C1)], exb.at[slot], wsems[slot]).wait()

  def compute(g, slot):
    @pl.when(g >= 2)
    def _():
      drain_write(slot)
    def edge(j, c):
      vs = tsb[slot, j, pl.ds(0, LANES)]
      vd = tdb[slot, j, pl.ds(0, LANES)]
      vdr = lax.gather(vd, rot_idx, _BCAST_DNUMS, (1,),
                       mode=lax.GatherScatterMode.PROMISE_IN_BOUNDS)
      al = vs + vdr
      al = jnp.maximum(al, al * jnp.float32(0.2))
      ex = jnp.exp(al)
      exb[slot, j] = ex
      exs[j, pl.ds(0, LANES)] = ex
      return c

    lax.fori_loop(0, C1, edge, 0)
    for off in (0, 16, 24):
      dstc[pl.ds(off, LANES)] = didxb[slot, pl.ds(off, LANES)]
    pltpu.sync_copy(exs, acc.at[dstc], add=True)
    pltpu.async_copy(exb.at[slot],
                     ex_hbm.at[pl.ds(ebase + g * C1, C1)], wsems[slot])

  issue_idx(0, 0)
  issue_idx(1, 1)
  wait_idx(0)
  issue_data(0)

  def body(i, carry):
    g0 = 2 * i
    wait_idx(1)
    issue_data(1)
    wait_data(0)
    compute(g0, 0)
    issue_idx(jnp.minimum(g0 + 2, G1 - 1), 0)
    wait_idx(0)
    issue_data(0)
    wait_data(1)
    compute(g0 + 1, 1)
    issue_idx(jnp.minimum(g0 + 3, G1 - 1), 1)
    return carry

  lax.fori_loop(0, G1 // 2, body, 0)
  wait_data(0)
  wait_idx(1)
  drain_write(0)
  drain_write(1)

  plsc.subcore_barrier()
  _split_copy(sid,
              lambda o, n: acc.at[pl.ds(o, n)],
              lambda o, n: dpart_hbm.at[cid, pl.ds(o, n)])


_sc_pass1 = pl.kernel(
    _sc_pass1_body,
    out_type=(jax.ShapeDtypeStruct((E, LANES), jnp.float32),
              jax.ShapeDtypeStruct((NC, N, HID), jnp.float32)),
    mesh=_SC_MESH,
    scratch_types=[
        pltpu.VMEM((2, C1), jnp.int32),
        pltpu.VMEM((2, C1), jnp.int32),
        pltpu.VMEM((C1,), jnp.int32),
        pltpu.VMEM((2, C1, HID), jnp.float32),
        pltpu.VMEM((2, C1, HID), jnp.float32),
        pltpu.VMEM((2, C1, LANES), jnp.float32),
        pltpu.VMEM((C1, HID), jnp.float32),
        pltpu.VMEM_SHARED((N, HID), jnp.float32),
        pltpu.SemaphoreType.DMA,
        pltpu.SemaphoreType.DMA,
        pltpu.SemaphoreType.DMA,
        pltpu.SemaphoreType.DMA,
        pltpu.SemaphoreType.DMA,
        pltpu.SemaphoreType.DMA,
    ],
)


def _dinv_body(d0_ref, d1_ref, t_ref, dinv_ref, cs_ref):
  t = t_ref[...]
  al = t[:, 0:HEADS] + t[:, HEADS:2 * HEADS]
  al = jnp.maximum(al, al * jnp.float32(0.2))
  exs = jnp.exp(al)
  den = d0_ref[...][:, 0:HEADS] + d1_ref[...][:, 0:HEADS] + exs
  dv = 1.0 / (den + jnp.float32(1e-16))
  blk = t.shape[0]
  dinv_ref[...] = jnp.concatenate(
      [dv, jnp.zeros((blk, HID - HEADS), jnp.float32)], axis=1)
  cs_ref[...] = exs * dv


def _dinv(d0, d1, t):
  blk = 2000
  spec = pl.BlockSpec((blk, HID), lambda i: (i, 0))
  return pl.pallas_call(
      _dinv_body,
      grid=(N // blk,),
      in_specs=[spec, spec, spec],
      out_specs=[spec, pl.BlockSpec((blk, HEADS), lambda i: (i, 0))],
      out_shape=[jax.ShapeDtypeStruct((N, HID), jnp.float32),
                 jax.ShapeDtypeStruct((N, HEADS), jnp.float32)],
  )(d0, d1, t)


def _sc_pass2_body(h_hbm, dinv_hbm, ex_hbm, src_hbm, dst_hbm, z_hbm,
                   mpart_hbm,
                   sidxb, didxb, dstc, hbuf, exb, dvb, msgb, acc,
                   isem0, isem1, hsem0, hsem1, esem0, esem1):
  cid = lax.axis_index("c")
  sid = lax.axis_index("s")
  wid = sid * NC + cid
  isems = (isem0, isem1)
  hsems = (hsem0, hsem1)
  esems = (esem0, esem1)

  _split_copy(sid,
              lambda o, n: z_hbm.at[pl.ds(o, n)],
              lambda o, n: acc.at[pl.ds(o, n)])
  plsc.subcore_barrier()

  ebase = wid * EPW

  def issue_idx(g, slot):
    pltpu.async_copy(src_hbm.at[pl.ds(ebase + g * C2, C2)], sidxb.at[slot],
                     isems[slot])
    pltpu.async_copy(dst_hbm.at[pl.ds(ebase + g * C2, C2)], didxb.at[slot],
                     isems[slot])

  def wait_idx(slot):
    pltpu.make_async_copy(
        src_hbm.at[pl.ds(0, C2)], sidxb.at[slot], isems[slot]).wait()
    pltpu.make_async_copy(
        src_hbm.at[pl.ds(0, C2)], didxb.at[slot], isems[slot]).wait()

  def issue_data(g, slot):
    pltpu.async_copy(h_hbm.at[sidxb.at[slot]], hbuf.at[slot], hsems[slot])
    pltpu.async_copy(dinv_hbm.at[didxb.at[slot]], dvb.at[slot], esems[slot])
    pltpu.async_copy(ex_hbm.at[pl.ds(ebase + g * C2, C2)], exb.at[slot],
                     esems[slot])

  def wait_data(slot):
    pltpu.make_async_copy(
        h_hbm.at[pl.ds(0, C2)], hbuf.at[slot], hsems[slot]).wait()
    pltpu.make_async_copy(
        dinv_hbm.at[pl.ds(0, C2)], dvb.at[slot], esems[slot]).wait()
    pltpu.make_async_copy(
        ex_hbm.at[pl.ds(0, C2)], exb.at[slot], esems[slot]).wait()

  def compute(g, slot):
    def edge_pair(jh, carry):
      for half in (0, 1):
        j = 2 * jh + half
        coef = exb[slot, j] * dvb[slot, j, pl.ds(0, LANES)]
        m = [None] * 8
        for hh in range(HEADS):
          c = _lane_bcast(coef, hh)
          for t2 in range(4):
            wi = (hh * 4 + t2) * LANES
            w = hbuf[slot, j, wi // HID, pl.ds(wi % HID, LANES)]
            va = lax.bitcast_convert_type(lax.shift_left(w, 16), jnp.float32)
            vb = lax.bitcast_convert_type(w & jnp.int32(-65536), jnp.float32)
            k = 2 * t2
            m[k] = c * va if hh == 0 else m[k] + c * va
            m[k + 1] = c * vb if hh == 0 else m[k + 1] + c * vb
        for k in range(8):
          msgb[j, pl.ds(k * LANES, LANES)] = m[k]
      return carry

    lax.fori_loop(0, C2 // 2, edge_pair, 0)
    dstc[pl.ds(0, C2)] = didxb[slot, pl.ds(0, C2)]
    pltpu.sync_copy(msgb, acc.at[dstc], add=True)

  issue_idx(0, 0)
  issue_idx(1, 1)
  wait_idx(0)
  issue_data(0, 0)

  def body(i, carry):
    g0 = 2 * i
    wait_idx(1)
    issue_data(g0 + 1, 1)
    wait_data(0)
    compute(g0, 0)
    issue_idx(jnp.minimum(g0 + 2, G2 - 1), 0)
    wait_idx(0)
    issue_data(jnp.minimum(g0 + 2, G2 - 1), 0)
    wait_data(1)
    compute(g0 + 1, 1)
    issue_idx(jnp.minimum(g0 + 3, G2 - 1), 1)
    return carry

  lax.fori_loop(0, G2 // 2, body, 0)
  wait_data(0)
  compute(G2 - 1, 0)
  wait_idx(1)

  plsc.subcore_barrier()
  _split_copy(sid,
              lambda o, n: acc.at[pl.ds(o, n)],
              lambda o, n: mpart_hbm.at[cid, pl.ds(o, n)])


_sc_pass2 = pl.kernel(
    _sc_pass2_body,
    out_type=jax.ShapeDtypeStruct((NC, N, HID), jnp.float32),
    mesh=_SC_MESH,
    scratch_types=[
        pltpu.VMEM((2, C2), jnp.int32),
        pltpu.VMEM((2, C2), jnp.int32),
        pltpu.VMEM((C2,), jnp.int32),
        pltpu.VMEM((2, C2, 4, HID), jnp.int32),
        pltpu.VMEM((2, C2, LANES), jnp.float32),
        pltpu.VMEM((2, C2, HID), jnp.float32),
        pltpu.VMEM((C2, HID), jnp.float32),
        pltpu.VMEM_SHARED((N, HID), jnp.float32),
        pltpu.SemaphoreType.DMA,
        pltpu.SemaphoreType.DMA,
        pltpu.SemaphoreType.DMA,
        pltpu.SemaphoreType.DMA,
        pltpu.SemaphoreType.DMA,
        pltpu.SemaphoreType.DMA,
    ],
)


def _fin_body(m0_ref, m1_ref, h_ref, cs_ref, b_ref, y_ref):
  blk = h_ref.shape[0]
  hb = h_ref[...].reshape(blk, HEADS, HID)
  cs = cs_ref[...]
  selfterm = (hb * cs[:, :, None]).sum(axis=1)
  y = (m0_ref[...] + m1_ref[...] + selfterm) * jnp.float32(1.0 / HEADS)
  y = y + b_ref[...]
  y_ref[...] = jnp.where(y > 0, y, jnp.exp(y) - 1.0)


def _fin(m0, m1, h, cs, b2d):
  blk = 400
  return pl.pallas_call(
      _fin_body,
      grid=(N // blk,),
      in_specs=[
          pl.BlockSpec((blk, HID), lambda i: (i, 0)),
          pl.BlockSpec((blk, HID), lambda i: (i, 0)),
          pl.BlockSpec((blk, HEADS * HID), lambda i: (i, 0)),
          pl.BlockSpec((blk, HEADS), lambda i: (i, 0)),
          pl.BlockSpec((1, HID), lambda i: (0, 0)),
      ],
      out_specs=pl.BlockSpec((blk, HID), lambda i: (i, 0)),
      out_shape=jax.ShapeDtypeStruct((N, HID), jnp.float32),
  )(m0, m1, h, cs, b2d)


def _pool_body(b_ref, y0_ref, y1_ref, y2_ref, o0_ref, o1_ref, o2_ref):
  i = pl.program_id(0)

  @pl.when(i == 0)
  def _():
    o0_ref[...] = jnp.zeros_like(o0_ref)
    o1_ref[...] = jnp.zeros_like(o1_ref)
    o2_ref[...] = jnp.zeros_like(o2_ref)

  b = b_ref[0, 0, :]
  blk = b.shape[0]
  onehot = (lax.broadcasted_iota(jnp.int32, (G, blk), 0)
            == b[None, :]).astype(jnp.float32)
  o0_ref[...] += jnp.dot(onehot, y0_ref[...],
                         preferred_element_type=jnp.float32)
  o1_ref[...] += jnp.dot(onehot, y1_ref[...],
                         preferred_element_type=jnp.float32)
  o2_ref[...] += jnp.dot(onehot, y2_ref[...],
                         preferred_element_type=jnp.float32)


def _pool(batch_r, y0, y1, y2):
  blk = 400
  yspec = pl.BlockSpec((blk, HID), lambda i: (i, 0))
  ospec = pl.BlockSpec((G, HID), lambda i: (0, 0))
  oshape = jax.ShapeDtypeStruct((G, HID), jnp.float32)
  return pl.pallas_call(
      _pool_body,
      grid=(N // blk,),
      in_specs=[pl.BlockSpec((1, 1, blk), lambda i: (i, 0, 0)),
                yspec, yspec, yspec],
      out_specs=[ospec, ospec, ospec],
      out_shape=[oshape, oshape, oshape],
  )(batch_r, y0, y1, y2)


def _tmat(att_s, att_d):
  eye = jnp.eye(HEADS, dtype=jnp.float32)
  ms = (att_s[0][:, :, None] * eye[:, None, :]).reshape(HEADS * HID, HEADS)
  md = (att_d[0][:, :, None] * eye[:, None, :]).reshape(HEADS * HID, HEADS)
  return jnp.pad(jnp.concatenate([ms, md], axis=1),
                 ((0, 0), (0, HID - 2 * HEADS)))


def kernel(x, edge_index, batch, edge_attr,
           W0, att_src0, att_dst0, bias0,
           W1, att_src1, att_dst1, bias1,
           W2, att_src2, att_dst2, bias2):
  del edge_attr
  src_r = edge_index[0].astype(jnp.int32)
  dst_r = edge_index[1].astype(jnp.int32)
  batch_r = batch.astype(jnp.int32).reshape(N // 400, 1, 400)
  z128 = jnp.zeros((N, HID), jnp.float32)

  params = [(W0, att_src0, att_dst0, bias0),
            (W1, att_src1, att_dst1, bias1),
            (W2, att_src2, att_dst2, bias2)]
  wi = np.arange(HEADS * HID // 2)
  t_blk, i_lane = wi // LANES, wi % LANES
  perm_lo = jnp.asarray(32 * t_blk + i_lane, jnp.int32)
  perm_hi = jnp.asarray(32 * t_blk + LANES + i_lane, jnp.int32)
  h_in = x
  ys = []
  for (W, a_s, a_d, b) in params:
    h, t, hpk = _mm(h_in, W, _tmat(a_s, a_d), W[:, perm_lo], W[:, perm_hi])
    ex, dpart = _sc_pass1(t, src_r, dst_r, z128)
    dinv, cself = _dinv(dpart[0], dpart[1], t)
    mpart = _sc_pass2(hpk, dinv, ex, src_r, dst_r, z128)
    y = _fin(mpart[0], mpart[1], h, cself, b.reshape(1, HID))
    ys.append(y)
    h_in = y

  rep0, rep1, rep2 = _pool(batch_r, ys[0], ys[1], ys[2])
  global_rep = jnp.concatenate([rep0, rep1, rep2], axis=1)
  return (global_rep, h_in)


# speedup vs baseline: 1.5994x; 1.0861x over previous
import functools

import numpy as np

import jax
import jax.numpy as jnp
from jax import lax
from jax.experimental import pallas as pl
from jax.experimental.pallas import tpu as pltpu
from jax.experimental.pallas import tpu_sc as plsc

N = 10000
E = 320000
FEAT = 128
HID = 128
HEADS = 8
G = 16
LANES = 16
NC = 2
NS = 16
NW = NC * NS
EPW = E // NW
C2 = 16
G2 = EPW // C2
C1 = 40
G1 = EPW // C1
RPT = 624
RPT_LAST = N - (NS - 1) * RPT


def _split_copy(sid, mk_src, mk_dst):
  off = sid * RPT

  @pl.when(sid < NS - 1)
  def _():
    pltpu.sync_copy(mk_src(off, RPT), mk_dst(off, RPT))

  @pl.when(sid == NS - 1)
  def _():
    pltpu.sync_copy(mk_src(off, RPT_LAST), mk_dst(off, RPT_LAST))

_SC_MESH = plsc.VectorSubcoreMesh(
    core_axis_name="c", subcore_axis_name="s", num_cores=NC, num_subcores=NS)

_BCAST_DNUMS = lax.GatherDimensionNumbers(
    offset_dims=(), collapsed_slice_dims=(0,), start_index_map=(0,))


def _lane_bcast(vec, lane):
  idx = jnp.full((LANES, 1), lane, jnp.int32)
  return lax.gather(vec, idx, _BCAST_DNUMS, (1,),
                    mode=lax.GatherScatterMode.PROMISE_IN_BOUNDS)


def _mm_body(x_ref, w_ref, t_ref, wlo_ref, whi_ref, h_ref, tout_ref,
             hpk_ref):
  x = x_ref[...]
  h = jnp.dot(x, w_ref[...], preferred_element_type=jnp.float32)
  h_ref[...] = h
  tout_ref[...] = jnp.dot(h, t_ref[...], preferred_element_type=jnp.float32)
  ha = jnp.dot(x, wlo_ref[...], preferred_element_type=jnp.float32)
  hb = jnp.dot(x, whi_ref[...], preferred_element_type=jnp.float32)
  au = lax.bitcast_convert_type(
      lax.convert_element_type(ha, jnp.bfloat16), jnp.uint16)
  bu = lax.bitcast_convert_type(
      lax.convert_element_type(hb, jnp.bfloat16), jnp.uint16)
  word = au.astype(jnp.uint32) | (bu.astype(jnp.uint32) << 16)
  blk = word.shape[0]
  hpk_ref[...] = lax.bitcast_convert_type(word, jnp.int32).reshape(
      blk, 4, HID)


def _mm(x, w, tmat, wlo, whi):
  n, k = x.shape
  m = w.shape[1]
  blk = 400
  return pl.pallas_call(
      _mm_body,
      grid=(n // blk,),
      in_specs=[
          pl.BlockSpec((blk, k), lambda i: (i, 0)),
          pl.BlockSpec((k, m), lambda i: (0, 0)),
          pl.BlockSpec((m, HID), lambda i: (0, 0)),
          pl.BlockSpec((k, m // 2), lambda i: (0, 0)),
          pl.BlockSpec((k, m // 2), lambda i: (0, 0)),
      ],
      out_specs=[
          pl.BlockSpec((blk, m), lambda i: (i, 0)),
          pl.BlockSpec((blk, HID), lambda i: (i, 0)),
          pl.BlockSpec((blk, 4, HID), lambda i: (i, 0, 0)),
      ],
      out_shape=[
          jax.ShapeDtypeStruct((n, m), jnp.float32),
          jax.ShapeDtypeStruct((n, HID), jnp.float32),
          jax.ShapeDtypeStruct((n, 4, HID), jnp.int32),
      ],
  )(x, w, tmat, wlo, whi)


def _sc_pass1_body(t_hbm, src_hbm, dst_hbm, z_hbm,
                   ex_hbm, dpart_hbm,
                   sidxb, didxb, dstc, tsb, tdb, exb, exs, acc,
                   isem0, isem1, gsem0, gsem1, wsem0, wsem1, ssem):
  cid = lax.axis_index("c")
  sid = lax.axis_index("s")
  wid = sid * NC + cid
  isems = (isem0, isem1)
  gsems = (gsem0, gsem1)
  wsems = (wsem0, wsem1)

  _split_copy(sid,
              lambda o, n: z_hbm.at[pl.ds(o, n)],
              lambda o, n: acc.at[pl.ds(o, n)])
  pltpu.sync_copy(z_hbm.at[pl.ds(0, C1)], exs.at[0])
  pltpu.sync_copy(z_hbm.at[pl.ds(0, C1)], exs.at[1])
  plsc.subcore_barrier()

  ebase = wid * EPW

  def drain_scatter():
    pltpu.make_async_copy(
        z_hbm.at[pl.ds(0, C1)], acc.at[pl.ds(0, C1)], ssem).wait()

  rot_idx = jnp.reshape((lax.iota(jnp.int32, LANES) + 8) & 15, (LANES, 1))

  def issue_idx(g, slot):
    pltpu.async_copy(src_hbm.at[pl.ds(ebase + g * C1, C1)], sidxb.at[slot],
                     isems[slot])
    pltpu.async_copy(dst_hbm.at[pl.ds(ebase + g * C1, C1)], didxb.at[slot],
                     isems[slot])

  def wait_idx(slot):
    pltpu.make_async_copy(
        src_hbm.at[pl.ds(0, C1)], sidxb.at[slot], isems[slot]).wait()
    pltpu.make_async_copy(
        src_hbm.at[pl.ds(0, C1)], didxb.at[slot], isems[slot]).wait()

  def issue_data(slot):
    pltpu.async_copy(t_hbm.at[sidxb.at[slot]], tsb.at[slot], gsems[slot])
    pltpu.async_copy(t_hbm.at[didxb.at[slot]], tdb.at[slot], gsems[slot])

  def wait_data(slot):
    pltpu.make_async_copy(
        t_hbm.at[pl.ds(0, C1)], tsb.at[slot], gsems[slot]).wait()
    pltpu.make_async_copy(
        t_hbm.at[pl.ds(0, C1)], tdb.at[slot], gsems[slot]).wait()

  def drain_write(slot):
    pltpu.make_async_copy(
        ex_hbm.at[pl.ds(0, C1)], exb.at[slot], wsems[slot]).wait()

  def compute(g, slot):
    @pl.when(g >= 2)
    def _():
      drain_write(slot)
    def edge(j, c):
      vs = tsb[slot, j, pl.ds(0, LANES)]
      vd = tdb[slot, j, pl.ds(0, LANES)]
      vdr = lax.gather(vd, rot_idx, _BCAST_DNUMS, (1,),
                       mode=lax.GatherScatterMode.PROMISE_IN_BOUNDS)
      al = vs + vdr
      al = jnp.maximum(al, al * jnp.float32(0.2))
      ex = jnp.exp(al)
      exb[slot, j] = ex
      exs[slot, j, pl.ds(0, LANES)] = ex
      return c

    @pl.when(g >= 2)
    def _():
      drain_scatter()
    lax.fori_loop(0, C1, edge, 0)
    for off in (0, 16, 24):
      dstc[slot, pl.ds(off, LANES)] = didxb[slot, pl.ds(off, LANES)]
    pltpu.async_copy(exs.at[slot], acc.at[dstc.at[slot]], ssem, add=True)
    pltpu.async_copy(exb.at[slot],
                     ex_hbm.at[pl.ds(ebase + g * C1, C1)], wsems[slot])

  issue_idx(0, 0)
  issue_idx(1, 1)
  wait_idx(0)
  issue_data(0)

  def body(i, carry):
    g0 = 2 * i
    wait_idx(1)
    issue_data(1)
    wait_data(0)
    compute(g0, 0)
    issue_idx(jnp.minimum(g0 + 2, G1 - 1), 0)
    wait_idx(0)
    issue_data(0)
    wait_data(1)
    compute(g0 + 1, 1)
    issue_idx(jnp.minimum(g0 + 3, G1 - 1), 1)
    return carry

  lax.fori_loop(0, G1 // 2, body, 0)
  wait_data(0)
  wait_idx(1)
  drain_write(0)
  drain_write(1)
  drain_scatter()
  drain_scatter()

  plsc.subcore_barrier()
  _split_copy(sid,
              lambda o, n: acc.at[pl.ds(o, n)],
              lambda o, n: dpart_hbm.at[cid, pl.ds(o, n)])


_sc_pass1 = pl.kernel(
    _sc_pass1_body,
    out_type=(jax.ShapeDtypeStruct((E, LANES), jnp.float32),
              jax.ShapeDtypeStruct((NC, N, HID), jnp.float32)),
    mesh=_SC_MESH,
    scratch_types=[
        pltpu.VMEM((2, C1), jnp.int32),
        pltpu.VMEM((2, C1), jnp.int32),
        pltpu.VMEM((2, C1), jnp.int32),
        pltpu.VMEM((2, C1, HID), jnp.float32),
        pltpu.VMEM((2, C1, HID), jnp.float32),
        pltpu.VMEM((2, C1, LANES), jnp.float32),
        pltpu.VMEM((2, C1, HID), jnp.float32),
        pltpu.VMEM_SHARED((N, HID), jnp.float32),
        pltpu.SemaphoreType.DMA,
        pltpu.SemaphoreType.DMA,
        pltpu.SemaphoreType.DMA,
        pltpu.SemaphoreType.DMA,
        pltpu.SemaphoreType.DMA,
        pltpu.SemaphoreType.DMA,
        pltpu.SemaphoreType.DMA,
    ],
)


def _dinv_body(d0_ref, d1_ref, t_ref, dinv_ref, cs_ref):
  t = t_ref[...]
  al = t[:, 0:HEADS] + t[:, HEADS:2 * HEADS]
  al = jnp.maximum(al, al * jnp.float32(0.2))
  exs = jnp.exp(al)
  den = d0_ref[...][:, 0:HEADS] + d1_ref[...][:, 0:HEADS] + exs
  dv = 1.0 / (den + jnp.float32(1e-16))
  blk = t.shape[0]
  dinv_ref[...] = jnp.concatenate(
      [dv, jnp.zeros((blk, HID - HEADS), jnp.float32)], axis=1)
  cs_ref[...] = exs * dv


def _dinv(d0, d1, t):
  blk = 2000
  spec = pl.BlockSpec((blk, HID), lambda i: (i, 0))
  return pl.pallas_call(
      _dinv_body,
      grid=(N // blk,),
      in_specs=[spec, spec, spec],
      out_specs=[spec, pl.BlockSpec((blk, HEADS), lambda i: (i, 0))],
      out_shape=[jax.ShapeDtypeStruct((N, HID), jnp.float32),
                 jax.ShapeDtypeStruct((N, HEADS), jnp.float32)],
  )(d0, d1, t)


def _sc_pass2_body(h_hbm, dinv_hbm, ex_hbm, src_hbm, dst_hbm, z_hbm,
                   mpart_hbm,
                   sidxb, didxb, dstc, hbuf, exb, dvb, msgb, acc,
                   isem0, isem1, hsem0, hsem1, esem0, esem1, ssem):
  cid = lax.axis_index("c")
  sid = lax.axis_index("s")
  wid = sid * NC + cid
  isems = (isem0, isem1)
  hsems = (hsem0, hsem1)
  esems = (esem0, esem1)

  _split_copy(sid,
              lambda o, n: z_hbm.at[pl.ds(o, n)],
              lambda o, n: acc.at[pl.ds(o, n)])
  plsc.subcore_barrier()

  ebase = wid * EPW

  def drain_scatter():
    pltpu.make_async_copy(
        z_hbm.at[pl.ds(0, C2)], acc.at[pl.ds(0, C2)], ssem).wait()

  def issue_idx(g, slot):
    pltpu.async_copy(src_hbm.at[pl.ds(ebase + g * C2, C2)], sidxb.at[slot],
                     isems[slot])
    pltpu.async_copy(dst_hbm.at[pl.ds(ebase + g * C2, C2)], didxb.at[slot],
                     isems[slot])

  def wait_idx(slot):
    pltpu.make_async_copy(
        src_hbm.at[pl.ds(0, C2)], sidxb.at[slot], isems[slot]).wait()
    pltpu.make_async_copy(
        src_hbm.at[pl.ds(0, C2)], didxb.at[slot], isems[slot]).wait()

  def issue_data(g, slot):
    pltpu.async_copy(h_hbm.at[sidxb.at[slot]], hbuf.at[slot], hsems[slot])
    pltpu.async_copy(dinv_hbm.at[didxb.at[slot]], dvb.at[slot], esems[slot])
    pltpu.async_copy(ex_hbm.at[pl.ds(ebase + g * C2, C2)], exb.at[slot],
                     esems[slot])

  def wait_data(slot):
    pltpu.make_async_copy(
        h_hbm.at[pl.ds(0, C2)], hbuf.at[slot], hsems[slot]).wait()
    pltpu.make_async_copy(
        dinv_hbm.at[pl.ds(0, C2)], dvb.at[slot], esems[slot]).wait()
    pltpu.make_async_copy(
        ex_hbm.at[pl.ds(0, C2)], exb.at[slot], esems[slot]).wait()

  def compute(g, slot):
    @pl.when(g >= 2)
    def _():
      drain_scatter()

    def edge_pair(jh, carry):
      for half in (0, 1):
        j = 2 * jh + half
        coef = exb[slot, j] * dvb[slot, j, pl.ds(0, LANES)]
        m = [None] * 8
        for hh in range(HEADS):
          c = _lane_bcast(coef, hh)
          for t2 in range(4):
            wi = (hh * 4 + t2) * LANES
            w = hbuf[slot, j, wi // HID, pl.ds(wi % HID, LANES)]
            va = lax.bitcast_convert_type(lax.shift_left(w, 16), jnp.float32)
            vb = lax.bitcast_convert_type(w & jnp.int32(-65536), jnp.float32)
            k = 2 * t2
            m[k] = c * va if hh == 0 else m[k] + c * va
            m[k + 1] = c * vb if hh == 0 else m[k + 1] + c * vb
        for k in range(8):
          msgb[slot, j, pl.ds(k * LANES, LANES)] = m[k]
      return carry

    lax.fori_loop(0, C2 // 2, edge_pair, 0)
    dstc[slot, pl.ds(0, C2)] = didxb[slot, pl.ds(0, C2)]
    pltpu.async_copy(msgb.at[slot], acc.at[dstc.at[slot]], ssem, add=True)

  issue_idx(0, 0)
  issue_idx(1, 1)
  wait_idx(0)
  issue_data(0, 0)

  def body(i, carry):
    g0 = 2 * i
    wait_idx(1)
    issue_data(g0 + 1, 1)
    wait_data(0)
    compute(g0, 0)
    issue_idx(jnp.minimum(g0 + 2, G2 - 1), 0)
    wait_idx(0)
    issue_data(jnp.minimum(g0 + 2, G2 - 1), 0)
    wait_data(1)
    compute(g0 + 1, 1)
    issue_idx(jnp.minimum(g0 + 3, G2 - 1), 1)
    return carry

  lax.fori_loop(0, G2 // 2, body, 0)
  wait_data(0)
  compute(G2 - 1, 0)
  wait_idx(1)
  drain_scatter()
  drain_scatter()

  plsc.subcore_barrier()
  _split_copy(sid,
              lambda o, n: acc.at[pl.ds(o, n)],
              lambda o, n: mpart_hbm.at[cid, pl.ds(o, n)])


_sc_pass2 = pl.kernel(
    _sc_pass2_body,
    out_type=jax.ShapeDtypeStruct((NC, N, HID), jnp.float32),
    mesh=_SC_MESH,
    scratch_types=[
        pltpu.VMEM((2, C2), jnp.int32),
        pltpu.VMEM((2, C2), jnp.int32),
        pltpu.VMEM((2, C2), jnp.int32),
        pltpu.VMEM((2, C2, 4, HID), jnp.int32),
        pltpu.VMEM((2, C2, LANES), jnp.float32),
        pltpu.VMEM((2, C2, HID), jnp.float32),
        pltpu.VMEM((2, C2, HID), jnp.float32),
        pltpu.VMEM_SHARED((N, HID), jnp.float32),
        pltpu.SemaphoreType.DMA,
        pltpu.SemaphoreType.DMA,
        pltpu.SemaphoreType.DMA,
        pltpu.SemaphoreType.DMA,
        pltpu.SemaphoreType.DMA,
        pltpu.SemaphoreType.DMA,
        pltpu.SemaphoreType.DMA,
    ],
)


def _fin_body(m0_ref, m1_ref, h_ref, cs_ref, b_ref, y_ref):
  blk = h_ref.shape[0]
  hb = h_ref[...].reshape(blk, HEADS, HID)
  cs = cs_ref[...]
  selfterm = (hb * cs[:, :, None]).sum(axis=1)
  y = (m0_ref[...] + m1_ref[...] + selfterm) * jnp.float32(1.0 / HEADS)
  y = y + b_ref[...]
  y_ref[...] = jnp.where(y > 0, y, jnp.exp(y) - 1.0)


def _fin(m0, m1, h, cs, b2d):
  blk = 400
  return pl.pallas_call(
      _fin_body,
      grid=(N // blk,),
      in_specs=[
          pl.BlockSpec((blk, HID), lambda i: (i, 0)),
          pl.BlockSpec((blk, HID), lambda i: (i, 0)),
          pl.BlockSpec((blk, HEADS * HID), lambda i: (i, 0)),
          pl.BlockSpec((blk, HEADS), lambda i: (i, 0)),
          pl.BlockSpec((1, HID), lambda i: (0, 0)),
      ],
      out_specs=pl.BlockSpec((blk, HID), lambda i: (i, 0)),
      out_shape=jax.ShapeDtypeStruct((N, HID), jnp.float32),
  )(m0, m1, h, cs, b2d)


def _pool_body(b_ref, y0_ref, y1_ref, y2_ref, o0_ref, o1_ref, o2_ref):
  i = pl.program_id(0)

  @pl.when(i == 0)
  def _():
    o0_ref[...] = jnp.zeros_like(o0_ref)
    o1_ref[...] = jnp.zeros_like(o1_ref)
    o2_ref[...] = jnp.zeros_like(o2_ref)

  b = b_ref[0, 0, :]
  blk = b.shape[0]
  onehot = (lax.broadcasted_iota(jnp.int32, (G, blk), 0)
            == b[None, :]).astype(jnp.float32)
  o0_ref[...] += jnp.dot(onehot, y0_ref[...],
                         preferred_element_type=jnp.float32)
  o1_ref[...] += jnp.dot(onehot, y1_ref[...],
                         preferred_element_type=jnp.float32)
  o2_ref[...] += jnp.dot(onehot, y2_ref[...],
                         preferred_element_type=jnp.float32)


def _pool(batch_r, y0, y1, y2):
  blk = 400
  yspec = pl.BlockSpec((blk, HID), lambda i: (i, 0))
  ospec = pl.BlockSpec((G, HID), lambda i: (0, 0))
  oshape = jax.ShapeDtypeStruct((G, HID), jnp.float32)
  return pl.pallas_call(
      _pool_body,
      grid=(N // blk,),
      in_specs=[pl.BlockSpec((1, 1, blk), lambda i: (i, 0, 0)),
                yspec, yspec, yspec],
      out_specs=[ospec, ospec, ospec],
      out_shape=[oshape, oshape, oshape],
  )(batch_r, y0, y1, y2)


def _tmat(att_s, att_d):
  eye = jnp.eye(HEADS, dtype=jnp.float32)
  ms = (att_s[0][:, :, None] * eye[:, None, :]).reshape(HEADS * HID, HEADS)
  md = (att_d[0][:, :, None] * eye[:, None, :]).reshape(HEADS * HID, HEADS)
  return jnp.pad(jnp.concatenate([ms, md], axis=1),
                 ((0, 0), (0, HID - 2 * HEADS)))


def kernel(x, edge_index, batch, edge_attr,
           W0, att_src0, att_dst0, bias0,
           W1, att_src1, att_dst1, bias1,
           W2, att_src2, att_dst2, bias2):
  del edge_attr
  src_r = edge_index[0].astype(jnp.int32)
  dst_r = edge_index[1].astype(jnp.int32)
  batch_r = batch.astype(jnp.int32).reshape(N // 400, 1, 400)
  z128 = jnp.zeros((N, HID), jnp.float32)

  params = [(W0, att_src0, att_dst0, bias0),
            (W1, att_src1, att_dst1, bias1),
            (W2, att_src2, att_dst2, bias2)]
  wi = np.arange(HEADS * HID // 2)
  t_blk, i_lane = wi // LANES, wi % LANES
  perm_lo = jnp.asarray(32 * t_blk + i_lane, jnp.int32)
  perm_hi = jnp.asarray(32 * t_blk + LANES + i_lane, jnp.int32)
  h_in = x
  ys = []
  for (W, a_s, a_d, b) in params:
    h, t, hpk = _mm(h_in, W, _tmat(a_s, a_d), W[:, perm_lo], W[:, perm_hi])
    ex, dpart = _sc_pass1(t, src_r, dst_r, z128)
    dinv, cself = _dinv(dpart[0], dpart[1], t)
    mpart = _sc_pass2(hpk, dinv, ex, src_r, dst_r, z128)
    y = _fin(mpart[0], mpart[1], h, cself, b.reshape(1, HID))
    ys.append(y)
    h_in = y

  rep0, rep1, rep2 = _pool(batch_r, ys[0], ys[1], ys[2])
  global_rep = jnp.concatenate([rep0, rep1, rep2], axis=1)
  return (global_rep, h_in)


# speedup vs baseline: 1.8817x; 1.1765x over previous
import functools

import numpy as np

import jax
import jax.numpy as jnp
from jax import lax
from jax.experimental import pallas as pl
from jax.experimental.pallas import tpu as pltpu
from jax.experimental.pallas import tpu_sc as plsc

N = 10000
E = 320000
FEAT = 128
HID = 128
HEADS = 8
G = 16
LANES = 16
NC = 2
NS = 16
NW = NC * NS
EPW = E // NW
C2 = 16
G2 = EPW // C2
C1 = 40
G1 = EPW // C1
RPT = 624
RPT_LAST = N - (NS - 1) * RPT


def _split_copy(sid, mk_src, mk_dst):
  off = sid * RPT

  @pl.when(sid < NS - 1)
  def _():
    pltpu.sync_copy(mk_src(off, RPT), mk_dst(off, RPT))

  @pl.when(sid == NS - 1)
  def _():
    pltpu.sync_copy(mk_src(off, RPT_LAST), mk_dst(off, RPT_LAST))

_SC_MESH = plsc.VectorSubcoreMesh(
    core_axis_name="c", subcore_axis_name="s", num_cores=NC, num_subcores=NS)

_BCAST_DNUMS = lax.GatherDimensionNumbers(
    offset_dims=(), collapsed_slice_dims=(0,), start_index_map=(0,))


def _lane_bcast(vec, lane):
  idx = jnp.full((LANES, 1), lane, jnp.int32)
  return lax.gather(vec, idx, _BCAST_DNUMS, (1,),
                    mode=lax.GatherScatterMode.PROMISE_IN_BOUNDS)


def _mm_body(x_ref, w_ref, t_ref, wlo_ref, whi_ref, h_ref, tout_ref,
             hpk_ref):
  x = x_ref[...]
  h = jnp.dot(x, w_ref[...], preferred_element_type=jnp.float32)
  h_ref[...] = h
  tout_ref[...] = jnp.dot(h, t_ref[...], preferred_element_type=jnp.float32)
  ha = jnp.dot(x, wlo_ref[...], preferred_element_type=jnp.float32)
  hb = jnp.dot(x, whi_ref[...], preferred_element_type=jnp.float32)
  au = lax.bitcast_convert_type(
      lax.convert_element_type(ha, jnp.bfloat16), jnp.uint16)
  bu = lax.bitcast_convert_type(
      lax.convert_element_type(hb, jnp.bfloat16), jnp.uint16)
  word = au.astype(jnp.uint32) | (bu.astype(jnp.uint32) << 16)
  blk = word.shape[0]
  hpk_ref[...] = lax.bitcast_convert_type(word, jnp.int32).reshape(
      blk, 4, HID)


def _mm(x, w, tmat, wlo, whi):
  n, k = x.shape
  m = w.shape[1]
  blk = 400
  return pl.pallas_call(
      _mm_body,
      grid=(n // blk,),
      in_specs=[
          pl.BlockSpec((blk, k), lambda i: (i, 0)),
          pl.BlockSpec((k, m), lambda i: (0, 0)),
          pl.BlockSpec((m, HID), lambda i: (0, 0)),
          pl.BlockSpec((k, m // 2), lambda i: (0, 0)),
          pl.BlockSpec((k, m // 2), lambda i: (0, 0)),
      ],
      out_specs=[
          pl.BlockSpec((blk, m), lambda i: (i, 0)),
          pl.BlockSpec((blk, HID), lambda i: (i, 0)),
          pl.BlockSpec((blk, 4, HID), lambda i: (i, 0, 0)),
      ],
      out_shape=[
          jax.ShapeDtypeStruct((n, m), jnp.float32),
          jax.ShapeDtypeStruct((n, HID), jnp.float32),
          jax.ShapeDtypeStruct((n, 4, HID), jnp.int32),
      ],
  )(x, w, tmat, wlo, whi)


def _sc_pass1_body(t_hbm, src_hbm, dst_hbm, z_hbm,
                   ex_hbm, dpart_hbm,
                   sidxb, didxb, dstc, tsb, tdb, exb, exs, acc,
                   isem0, isem1, gsem0, gsem1, wsem0, wsem1, ssem):
  cid = lax.axis_index("c")
  sid = lax.axis_index("s")
  wid = sid * NC + cid
  isems = (isem0, isem1)
  gsems = (gsem0, gsem1)
  wsems = (wsem0, wsem1)

  _split_copy(sid,
              lambda o, n: z_hbm.at[pl.ds(o, n)],
              lambda o, n: acc.at[pl.ds(o, n)])
  pltpu.sync_copy(z_hbm.at[pl.ds(0, C1)], exs.at[0])
  pltpu.sync_copy(z_hbm.at[pl.ds(0, C1)], exs.at[1])
  plsc.subcore_barrier()

  ebase = wid * EPW

  def drain_scatter():
    pltpu.make_async_copy(
        z_hbm.at[pl.ds(0, C1)], acc.at[pl.ds(0, C1)], ssem).wait()

  rot_idx = jnp.reshape((lax.iota(jnp.int32, LANES) + 8) & 15, (LANES, 1))

  def issue_idx(g, slot):
    pltpu.async_copy(src_hbm.at[pl.ds(ebase + g * C1, C1)], sidxb.at[slot],
                     isems[slot])
    pltpu.async_copy(dst_hbm.at[pl.ds(ebase + g * C1, C1)], didxb.at[slot],
                     isems[slot])

  def wait_idx(slot):
    pltpu.make_async_copy(
        src_hbm.at[pl.ds(0, C1)], sidxb.at[slot], isems[slot]).wait()
    pltpu.make_async_copy(
        src_hbm.at[pl.ds(0, C1)], didxb.at[slot], isems[slot]).wait()

  def issue_data(slot):
    pltpu.async_copy(t_hbm.at[sidxb.at[slot]], tsb.at[slot], gsems[slot])
    pltpu.async_copy(t_hbm.at[didxb.at[slot]], tdb.at[slot], gsems[slot])

  def wait_data(slot):
    pltpu.make_async_copy(
        t_hbm.at[pl.ds(0, C1)], tsb.at[slot], gsems[slot]).wait()
    pltpu.make_async_copy(
        t_hbm.at[pl.ds(0, C1)], tdb.at[slot], gsems[slot]).wait()

  def drain_write(slot):
    pltpu.make_async_copy(
        ex_hbm.at[pl.ds(0, C1)], exb.at[slot], wsems[slot]).wait()

  def compute(g, gnext, slot):
    @pl.when(g >= 2)
    def _():
      drain_write(slot)
      drain_scatter()
    for off in (0, 16, 24):
      dstc[slot, pl.ds(off, LANES)] = didxb[slot, pl.ds(off, LANES)]
    issue_idx(gnext, slot)

    def edge(j, c):
      vs = tsb[slot, j, pl.ds(0, LANES)]
      vd = tdb[slot, j, pl.ds(0, LANES)]
      vdr = lax.gather(vd, rot_idx, _BCAST_DNUMS, (1,),
                       mode=lax.GatherScatterMode.PROMISE_IN_BOUNDS)
      al = vs + vdr
      al = jnp.maximum(al, al * jnp.float32(0.2))
      ex = jnp.exp(al)
      exb[slot, j] = ex
      exs[slot, j, pl.ds(0, LANES)] = ex
      return c

    lax.fori_loop(0, C1, edge, 0)
    pltpu.async_copy(exs.at[slot], acc.at[dstc.at[slot]], ssem, add=True)
    pltpu.async_copy(exb.at[slot],
                     ex_hbm.at[pl.ds(ebase + g * C1, C1)], wsems[slot])

  issue_idx(0, 0)
  issue_idx(1, 1)
  wait_idx(0)
  issue_data(0)

  def body(i, carry):
    g0 = 2 * i
    wait_idx(1)
    issue_data(1)
    wait_data(0)
    compute(g0, jnp.minimum(g0 + 2, G1 - 1), 0)
    wait_idx(0)
    issue_data(0)
    wait_data(1)
    compute(g0 + 1, jnp.minimum(g0 + 3, G1 - 1), 1)
    return carry

  lax.fori_loop(0, G1 // 2, body, 0)
  wait_data(0)
  wait_idx(1)
  drain_write(0)
  drain_write(1)
  drain_scatter()
  drain_scatter()

  plsc.subcore_barrier()
  _split_copy(sid,
              lambda o, n: acc.at[pl.ds(o, n)],
              lambda o, n: dpart_hbm.at[cid, pl.ds(o, n)])


_sc_pass1 = pl.kernel(
    _sc_pass1_body,
    out_type=(jax.ShapeDtypeStruct((E, LANES), jnp.float32),
              jax.ShapeDtypeStruct((NC, N, HID), jnp.float32)),
    mesh=_SC_MESH,
    scratch_types=[
        pltpu.VMEM((2, C1), jnp.int32),
        pltpu.VMEM((2, C1), jnp.int32),
        pltpu.VMEM((2, C1), jnp.int32),
        pltpu.VMEM((2, C1, HID), jnp.float32),
        pltpu.VMEM((2, C1, HID), jnp.float32),
        pltpu.VMEM((2, C1, LANES), jnp.float32),
        pltpu.VMEM((2, C1, HID), jnp.float32),
        pltpu.VMEM_SHARED((N, HID), jnp.float32),
        pltpu.SemaphoreType.DMA,
        pltpu.SemaphoreType.DMA,
        pltpu.SemaphoreType.DMA,
        pltpu.SemaphoreType.DMA,
        pltpu.SemaphoreType.DMA,
        pltpu.SemaphoreType.DMA,
        pltpu.SemaphoreType.DMA,
    ],
)


def _dinv_body(d0_ref, d1_ref, t_ref, dinv_ref, cs_ref):
  t = t_ref[...]
  al = t[:, 0:HEADS] + t[:, HEADS:2 * HEADS]
  al = jnp.maximum(al, al * jnp.float32(0.2))
  exs = jnp.exp(al)
  den = d0_ref[...][:, 0:HEADS] + d1_ref[...][:, 0:HEADS] + exs
  dv = 1.0 / (den + jnp.float32(1e-16))
  blk = t.shape[0]
  dinv_ref[...] = jnp.concatenate(
      [dv, jnp.zeros((blk, HID - HEADS), jnp.float32)], axis=1)
  cs_ref[...] = exs * dv


def _dinv(d0, d1, t):
  blk = 2000
  spec = pl.BlockSpec((blk, HID), lambda i: (i, 0))
  return pl.pallas_call(
      _dinv_body,
      grid=(N // blk,),
      in_specs=[spec, spec, spec],
      out_specs=[spec, pl.BlockSpec((blk, HEADS), lambda i: (i, 0))],
      out_shape=[jax.ShapeDtypeStruct((N, HID), jnp.float32),
                 jax.ShapeDtypeStruct((N, HEADS), jnp.float32)],
  )(d0, d1, t)


def _sc_pass2_body(h_hbm, dinv_hbm, ex_hbm, src_hbm, dst_hbm, z_hbm,
                   mpart_hbm,
                   sidxb, didxb, dstc, hbuf, exb, dvb, msgb, acc,
                   isem0, isem1, hsem0, hsem1, esem0, esem1, ssem):
  cid = lax.axis_index("c")
  sid = lax.axis_index("s")
  wid = sid * NC + cid
  isems = (isem0, isem1)
  hsems = (hsem0, hsem1)
  esems = (esem0, esem1)

  _split_copy(sid,
              lambda o, n: z_hbm.at[pl.ds(o, n)],
              lambda o, n: acc.at[pl.ds(o, n)])
  plsc.subcore_barrier()

  ebase = wid * EPW

  def drain_scatter():
    pltpu.make_async_copy(
        z_hbm.at[pl.ds(0, C2)], acc.at[pl.ds(0, C2)], ssem).wait()

  def issue_idx(g, slot):
    pltpu.async_copy(src_hbm.at[pl.ds(ebase + g * C2, C2)], sidxb.at[slot],
                     isems[slot])
    pltpu.async_copy(dst_hbm.at[pl.ds(ebase + g * C2, C2)], didxb.at[slot],
                     isems[slot])

  def wait_idx(slot):
    pltpu.make_async_copy(
        src_hbm.at[pl.ds(0, C2)], sidxb.at[slot], isems[slot]).wait()
    pltpu.make_async_copy(
        src_hbm.at[pl.ds(0, C2)], didxb.at[slot], isems[slot]).wait()

  def issue_data(g, slot):
    pltpu.async_copy(h_hbm.at[sidxb.at[slot]], hbuf.at[slot], hsems[slot])
    pltpu.async_copy(dinv_hbm.at[didxb.at[slot]], dvb.at[slot], esems[slot])
    pltpu.async_copy(ex_hbm.at[pl.ds(ebase + g * C2, C2)], exb.at[slot],
                     esems[slot])

  def wait_data(slot):
    pltpu.make_async_copy(
        h_hbm.at[pl.ds(0, C2)], hbuf.at[slot], hsems[slot]).wait()
    pltpu.make_async_copy(
        dinv_hbm.at[pl.ds(0, C2)], dvb.at[slot], esems[slot]).wait()
    pltpu.make_async_copy(
        ex_hbm.at[pl.ds(0, C2)], exb.at[slot], esems[slot]).wait()

  def compute(g, gnext, slot):
    @pl.when(g >= 2)
    def _():
      drain_scatter()
    dstc[slot, pl.ds(0, C2)] = didxb[slot, pl.ds(0, C2)]
    issue_idx(gnext, slot)

    def edge_pair(jh, carry):
      for half in (0, 1):
        j = 2 * jh + half
        coef = exb[slot, j] * dvb[slot, j, pl.ds(0, LANES)]
        m = [None] * 8
        for hh in range(HEADS):
          c = _lane_bcast(coef, hh)
          for t2 in range(4):
            wi = (hh * 4 + t2) * LANES
            w = hbuf[slot, j, wi // HID, pl.ds(wi % HID, LANES)]
            va = lax.bitcast_convert_type(lax.shift_left(w, 16), jnp.float32)
            vb = lax.bitcast_convert_type(w & jnp.int32(-65536), jnp.float32)
            k = 2 * t2
            m[k] = c * va if hh == 0 else m[k] + c * va
            m[k + 1] = c * vb if hh == 0 else m[k + 1] + c * vb
        for k in range(8):
          msgb[slot, j, pl.ds(k * LANES, LANES)] = m[k]
      return carry

    lax.fori_loop(0, C2 // 2, edge_pair, 0)
    pltpu.async_copy(msgb.at[slot], acc.at[dstc.at[slot]], ssem, add=True)

  issue_idx(0, 0)
  issue_idx(1, 1)
  wait_idx(0)
  issue_data(0, 0)

  def body(i, carry):
    g0 = 2 * i
    wait_idx(1)
    issue_data(g0 + 1, 1)
    wait_data(0)
    compute(g0, jnp.minimum(g0 + 2, G2 - 1), 0)
    wait_idx(0)
    issue_data(jnp.minimum(g0 + 2, G2 - 1), 0)
    wait_data(1)
    compute(g0 + 1, jnp.minimum(g0 + 3, G2 - 1), 1)
    return carry

  lax.fori_loop(0, G2 // 2, body, 0)
  wait_data(0)
  compute(G2 - 1, G2 - 1, 0)
  wait_idx(1)
  wait_idx(0)
  drain_scatter()
  drain_scatter()

  plsc.subcore_barrier()
  _split_copy(sid,
              lambda o, n: acc.at[pl.ds(o, n)],
              lambda o, n: mpart_hbm.at[cid, pl.ds(o, n)])


_sc_pass2 = pl.kernel(
    _sc_pass2_body,
    out_type=jax.ShapeDtypeStruct((NC, N, HID), jnp.float32),
    mesh=_SC_MESH,
    scratch_types=[
        pltpu.VMEM((2, C2), jnp.int32),
        pltpu.VMEM((2, C2), jnp.int32),
        pltpu.VMEM((2, C2), jnp.int32),
        pltpu.VMEM((2, C2, 4, HID), jnp.int32),
        pltpu.VMEM((2, C2, LANES), jnp.float32),
        pltpu.VMEM((2, C2, HID), jnp.float32),
        pltpu.VMEM((2, C2, HID), jnp.float32),
        pltpu.VMEM_SHARED((N, HID), jnp.float32),
        pltpu.SemaphoreType.DMA,
        pltpu.SemaphoreType.DMA,
        pltpu.SemaphoreType.DMA,
        pltpu.SemaphoreType.DMA,
        pltpu.SemaphoreType.DMA,
        pltpu.SemaphoreType.DMA,
        pltpu.SemaphoreType.DMA,
    ],
)


def _fin_body(m0_ref, m1_ref, h_ref, cs_ref, b_ref, y_ref):
  blk = h_ref.shape[0]
  hb = h_ref[...].reshape(blk, HEADS, HID)
  cs = cs_ref[...]
  selfterm = (hb * cs[:, :, None]).sum(axis=1)
  y = (m0_ref[...] + m1_ref[...] + selfterm) * jnp.float32(1.0 / HEADS)
  y = y + b_ref[...]
  y_ref[...] = jnp.where(y > 0, y, jnp.exp(y) - 1.0)


def _fin(m0, m1, h, cs, b2d):
  blk = 400
  return pl.pallas_call(
      _fin_body,
      grid=(N // blk,),
      in_specs=[
          pl.BlockSpec((blk, HID), lambda i: (i, 0)),
          pl.BlockSpec((blk, HID), lambda i: (i, 0)),
          pl.BlockSpec((blk, HEADS * HID), lambda i: (i, 0)),
          pl.BlockSpec((blk, HEADS), lambda i: (i, 0)),
          pl.BlockSpec((1, HID), lambda i: (0, 0)),
      ],
      out_specs=pl.BlockSpec((blk, HID), lambda i: (i, 0)),
      out_shape=jax.ShapeDtypeStruct((N, HID), jnp.float32),
  )(m0, m1, h, cs, b2d)


def _pool_body(b_ref, y0_ref, y1_ref, y2_ref, o0_ref, o1_ref, o2_ref):
  i = pl.program_id(0)

  @pl.when(i == 0)
  def _():
    o0_ref[...] = jnp.zeros_like(o0_ref)
    o1_ref[...] = jnp.zeros_like(o1_ref)
    o2_ref[...] = jnp.zeros_like(o2_ref)

  b = b_ref[0, 0, :]
  blk = b.shape[0]
  onehot = (lax.broadcasted_iota(jnp.int32, (G, blk), 0)
            == b[None, :]).astype(jnp.float32)
  o0_ref[...] += jnp.dot(onehot, y0_ref[...],
                         preferred_element_type=jnp.float32)
  o1_ref[...] += jnp.dot(onehot, y1_ref[...],
                         preferred_element_type=jnp.float32)
  o2_ref[...] += jnp.dot(onehot, y2_ref[...],
                         preferred_element_type=jnp.float32)


def _pool(batch_r, y0, y1, y2):
  blk = 400
  yspec = pl.BlockSpec((blk, HID), lambda i: (i, 0))
  ospec = pl.BlockSpec((G, HID), lambda i: (0, 0))
  oshape = jax.ShapeDtypeStruct((G, HID), jnp.float32)
  return pl.pallas_call(
      _pool_body,
      grid=(N // blk,),
      in_specs=[pl.BlockSpec((1, 1, blk), lambda i: (i, 0, 0)),
                yspec, yspec, yspec],
      out_specs=[ospec, ospec, ospec],
      out_shape=[oshape, oshape, oshape],
  )(batch_r, y0, y1, y2)


def _tmat(att_s, att_d):
  eye = jnp.eye(HEADS, dtype=jnp.float32)
  ms = (att_s[0][:, :, None] * eye[:, None, :]).reshape(HEADS * HID, HEADS)
  md = (att_d[0][:, :, None] * eye[:, None, :]).reshape(HEADS * HID, HEADS)
  return jnp.pad(jnp.concatenate([ms, md], axis=1),
                 ((0, 0), (0, HID - 2 * HEADS)))


def kernel(x, edge_index, batch, edge_attr,
           W0, att_src0, att_dst0, bias0,
           W1, att_src1, att_dst1, bias1,
           W2, att_src2, att_dst2, bias2):
  del edge_attr
  src_r = edge_index[0].astype(jnp.int32)
  dst_r = edge_index[1].astype(jnp.int32)
  batch_r = batch.astype(jnp.int32).reshape(N // 400, 1, 400)
  z128 = jnp.zeros((N, HID), jnp.float32)

  params = [(W0, att_src0, att_dst0, bias0),
            (W1, att_src1, att_dst1, bias1),
            (W2, att_src2, att_dst2, bias2)]
  wi = np.arange(HEADS * HID // 2)
  t_blk, i_lane = wi // LANES, wi % LANES
  perm_lo = jnp.asarray(32 * t_blk + i_lane, jnp.int32)
  perm_hi = jnp.asarray(32 * t_blk + LANES + i_lane, jnp.int32)
  h_in = x
  ys = []
  for (W, a_s, a_d, b) in params:
    h, t, hpk = _mm(h_in, W, _tmat(a_s, a_d), W[:, perm_lo], W[:, perm_hi])
    ex, dpart = _sc_pass1(t, src_r, dst_r, z128)
    dinv, cself = _dinv(dpart[0], dpart[1], t)
    mpart = _sc_pass2(hpk, dinv, ex, src_r, dst_r, z128)
    y = _fin(mpart[0], mpart[1], h, cself, b.reshape(1, HID))
    ys.append(y)
    h_in = y

  rep0, rep1, rep2 = _pool(batch_r, ys[0], ys[1], ys[2])
  global_rep = jnp.concatenate([rep0, rep1, rep2], axis=1)
  return (global_rep, h_in)
